# baseline (speedup 1.0000x reference)
import functools

import jax
import jax.numpy as jnp
from jax import lax
from jax.experimental import pallas as pl
from jax.experimental.pallas import tpu as pltpu
from jax.experimental.pallas import tpu_sc as plsc

_NUM_ROWS = 16384 * 20
_DIM = 64
_NW = 32
_B_PER_W = _NUM_ROWS // _NW
_CHUNK = 512
_NCHUNK = _B_PER_W // _CHUNK


def _gather_body(idx_hbm, table_hbm, out_hbm, idx_v, rows_v, sem):
    wid = lax.axis_index("s") * 2 + lax.axis_index("c")
    base = wid * _B_PER_W

    def chunk(i, carry):
        off = base + i * _CHUNK
        pltpu.sync_copy(idx_hbm.at[pl.ds(off, _CHUNK)], idx_v)
        pltpu.async_copy(table_hbm.at[idx_v], rows_v, sem).wait()
        pltpu.sync_copy(rows_v, out_hbm.at[pl.ds(off, _CHUNK)])
        return carry

    lax.fori_loop(0, _NCHUNK, chunk, 0)


_gather = functools.partial(
    pl.kernel,
    mesh=plsc.VectorSubcoreMesh(core_axis_name="c", subcore_axis_name="s"),
    compiler_params=pltpu.CompilerParams(use_tc_tiling_on_sc=False),
    out_type=jax.ShapeDtypeStruct((_NUM_ROWS, _DIM), jnp.float32),
    scratch_types=[
        pltpu.VMEM((_CHUNK,), jnp.int32),
        pltpu.VMEM((_CHUNK, _DIM), jnp.float32),
        pltpu.SemaphoreType.DMA,
    ],
)(_gather_body)


@jax.jit
def kernel(indexes, entity_table):
    flat_idx = indexes.reshape(-1)
    out = _gather(flat_idx, entity_table)
    return out.reshape(indexes.shape[0], indexes.shape[1], _DIM)


# speedup vs baseline: 1.0202x; 1.0202x over previous
import functools

import jax
import jax.numpy as jnp
from jax import lax
from jax.experimental import pallas as pl
from jax.experimental.pallas import tpu as pltpu
from jax.experimental.pallas import tpu_sc as plsc

_NUM_ROWS = 16384 * 20
_DIM = 64
_NW = 32
_B_PER_W = _NUM_ROWS // _NW
_CHUNK = 640
_NCHUNK = _B_PER_W // _CHUNK


def _gather_body(idx_hbm, table_hbm, out_hbm, idx_v, rows_v, sem_g, sem_w):
    wid = lax.axis_index("s") * 2 + lax.axis_index("c")
    base = wid * _B_PER_W

    pltpu.sync_copy(idx_hbm.at[wid], idx_v)

    gathers = [None] * _NCHUNK
    wbs = [None] * _NCHUNK
    gathers[0] = pltpu.async_copy(table_hbm.at[idx_v.at[0]], rows_v.at[0], sem_g)
    for i in range(1, _NCHUNK):
        if i >= 2:
            wbs[i - 2].wait()
        gathers[i] = pltpu.async_copy(
            table_hbm.at[idx_v.at[i]], rows_v.at[i % 2], sem_g)
        gathers[i - 1].wait()
        wbs[i - 1] = pltpu.async_copy(
            rows_v.at[(i - 1) % 2],
            out_hbm.at[pl.ds(base + (i - 1) * _CHUNK, _CHUNK)], sem_w)
    gathers[_NCHUNK - 1].wait()
    wbs[_NCHUNK - 1] = pltpu.async_copy(
        rows_v.at[(_NCHUNK - 1) % 2],
        out_hbm.at[pl.ds(base + (_NCHUNK - 1) * _CHUNK, _CHUNK)], sem_w)
    wbs[_NCHUNK - 2].wait()
    wbs[_NCHUNK - 1].wait()


_gather = functools.partial(
    pl.kernel,
    mesh=plsc.VectorSubcoreMesh(core_axis_name="c", subcore_axis_name="s"),
    compiler_params=pltpu.CompilerParams(use_tc_tiling_on_sc=False),
    out_type=jax.ShapeDtypeStruct((_NUM_ROWS, _DIM), jnp.float32),
    scratch_types=[
        pltpu.VMEM((_NCHUNK, _CHUNK), jnp.int32),
        pltpu.VMEM((2, _CHUNK, _DIM), jnp.float32),
        pltpu.SemaphoreType.DMA,
        pltpu.SemaphoreType.DMA,
    ],
)(_gather_body)


@jax.jit
def kernel(indexes, entity_table):
    flat_idx = indexes.reshape(_NW, _NCHUNK, _CHUNK)
    out = _gather(flat_idx, entity_table)
    return out.reshape(indexes.shape[0], indexes.shape[1], _DIM)
